# baseline (speedup 1.0000x reference)
import functools

import jax
import jax.numpy as jnp
import numpy as np
from jax import lax
from jax.experimental import pallas as pl
from jax.experimental.pallas import tpu as pltpu
from jax.experimental.pallas import tpu_sc as plsc

N = 10000
E = 320000
IN = 128
H = 4
D = 32
HD = H * D
NT = 4
ET = 8
SQRT_D = float(np.sqrt(D))
LAM = 0.3

KW = ET * HD
WBW = 2 * KW + 2 * HD
ACCW = 144

NWORK = 32
EPW = E // NWORK
CH = 80
NCH = EPW // CH
NPAD = 10240
ROWS_PT = NPAD // 16
ZR = 128

_f32 = jnp.float32


def _prep_body(wk_ref, wv_ref, wq_ref, wpre_ref, ba_ref, bm_ref, out_ref):
    for t in range(NT):
        katt = jnp.dot(wk_ref[t], ba_ref[...], preferred_element_type=_f32)
        vmsg = jnp.dot(wv_ref[t], bm_ref[...], preferred_element_type=_f32)
        for et in range(ET):
            out_ref[t, :, et * 2 * HD:et * 2 * HD + HD] = (
                katt[:, et * HD:(et + 1) * HD])
            out_ref[t, :, et * 2 * HD + HD:(et + 1) * 2 * HD] = (
                vmsg[:, et * HD:(et + 1) * HD])
        out_ref[t, :, 2 * KW:2 * KW + HD] = wq_ref[t]
        out_ref[t, :, 2 * KW + HD:WBW] = wpre_ref[t]


def _prep_weights(Wk, Wv, Wq, Wpre, big_att, big_msg):
    return pl.pallas_call(
        _prep_body,
        out_shape=jax.ShapeDtypeStruct((NT, IN, WBW), _f32),
    )(Wk, Wv, Wq, Wpre, big_att, big_msg)


_RBLK = 400


def _proj_body(x_ref, nt_ref, w_ref, khat_ref, q_ref, pre_ref):
    x = x_ref[...]
    nt = nt_ref[...]
    acc = jnp.zeros((_RBLK, WBW), dtype=_f32)
    for t in range(NT):
        p = jnp.dot(x, w_ref[t], preferred_element_type=_f32)
        acc = acc + jnp.where(nt == t, p, 0.0)
    khat_ref[...] = acc[:, 0:2 * KW].astype(jnp.bfloat16)
    q_ref[...] = acc[:, 2 * KW:2 * KW + HD]
    pre_ref[...] = acc[:, 2 * KW + HD:WBW]


def _project(x, ntype2d, wbig):
    grid = N // _RBLK
    return pl.pallas_call(
        _proj_body,
        grid=(grid,),
        in_specs=[
            pl.BlockSpec((_RBLK, IN), lambda i: (i, 0)),
            pl.BlockSpec((_RBLK, 1), lambda i: (i, 0)),
            pl.BlockSpec((NT, IN, WBW), lambda i: (0, 0, 0)),
        ],
        out_specs=[
            pl.BlockSpec((_RBLK, 2 * KW), lambda i: (i, 0)),
            pl.BlockSpec((_RBLK, HD), lambda i: (i, 0)),
            pl.BlockSpec((_RBLK, HD), lambda i: (i, 0)),
        ],
        out_shape=[
            jax.ShapeDtypeStruct((N, 2 * KW), jnp.bfloat16),
            jax.ShapeDtypeStruct((N, HD), _f32),
            jax.ShapeDtypeStruct((N, HD), _f32),
        ],
    )(x, ntype2d, wbig)


DENW = NPAD * H


def _edge_body(src_hbm, dst_hbm, et_hbm, kv_hbm, q_hbm,
               msg_hbm, ex_hbm,
               acc_sh, src_v, dst_v, et_v, idx_v, kv_v, q_v, m_v,
               ex_v, esem, dsem, gsem, ssem, xsem):
    cid = lax.axis_index("c")
    sid = lax.axis_index("s")
    wid = cid * 16 + sid
    iota16 = lax.iota(jnp.int32, 16)
    rows_sl = pl.ds(sid * ROWS_PT, ROWS_PT)

    def _zero_mv(r, _):
        for c in range(HD // 16):
            m_v[r, pl.ds(c * 16, 16)] = jnp.zeros((16,), _f32)
        return 0

    lax.fori_loop(0, CH, _zero_mv, 0)
    for j in range(ROWS_PT // CH):
        pltpu.sync_copy(m_v, acc_sh.at[pl.ds(sid * ROWS_PT + j * CH, CH)])
    plsc.subcore_barrier()

    def _eslice(hbm, ck, buf, sem):
        e0 = wid * EPW + ck * CH
        return pltpu.make_async_copy(hbm.at[pl.ds(e0, CH)], buf, sem)

    def _exslice(ck, sem):
        e0 = wid * EPW + ck * CH
        return pltpu.make_async_copy(ex_v, ex_hbm.at[pl.ds(e0 * H, CH * H)],
                                     sem)

    _eslice(src_hbm, 0, src_v, esem).start()
    _eslice(et_hbm, 0, et_v, esem).start()

    def _chunk(ck, _):
        @pl.when(ck > 0)
        def _():
            pltpu.make_async_copy(m_v, acc_sh.at[dst_v], ssem).wait()
            _exslice(ck - 1, xsem).wait()
        _eslice(dst_hbm, ck, dst_v, dsem).start()
        _eslice(src_hbm, ck, src_v, esem).wait()
        _eslice(et_hbm, ck, et_v, esem).wait()
        for g in range(CH // 16):
            s = src_v[pl.ds(g * 16, 16)]
            t = et_v[pl.ds(g * 16, 16)]
            idx_v[pl.ds(g * 16, 16)] = s * ET + t
        pltpu.async_copy(kv_hbm.at[idx_v], kv_v, gsem)
        _eslice(dst_hbm, ck, dst_v, dsem).wait()
        pltpu.async_copy(q_hbm.at[dst_v], q_v, gsem)
        pltpu.make_async_copy(kv_hbm.at[idx_v], kv_v, gsem).wait()
        pltpu.make_async_copy(q_hbm.at[dst_v], q_v, gsem).wait()

        iotaH = iota16 * H
        z16 = jnp.zeros((16,), _f32)
        W2 = D // 2
        for g in range(CH // 16):
            rows = g * 16 + iota16
            for h in range(H):
                colk0 = jnp.full((16,), h * W2, jnp.int32)

                def _unp(w):
                    return plsc.unpack(plsc.bitcast(w, jnp.bfloat16),
                                       format=plsc.PackFormat.INTERLEAVED,
                                       preferred_element_type=_f32)

                colq0 = jnp.full((16,), h * D, jnp.int32)

                def _dot(d, carry):
                    a0, a1, colk, colq = carry
                    kw = plsc.load_gather(kv_v, [rows, colk])
                    k0, k1 = _unp(kw)
                    qv0 = plsc.load_gather(q_v, [rows, colq])
                    qv1 = plsc.load_gather(q_v, [rows, colq + 1])
                    return (a0 + k0 * qv0, a1 + k1 * qv1, colk + 1, colq + 2)

                a0, a1, _, _ = lax.fori_loop(0, W2, _dot,
                                             (z16, z16, colk0, colq0),
                                             unroll=8)
                ex = jnp.exp(jnp.minimum(a0 + a1, 60.0))
                plsc.store_scatter(ex_v, [iotaH + (g * 16 * H + h)], ex)

                colv0 = jnp.full((16,), KVW2 + h * W2, jnp.int32)
                colm0 = jnp.full((16,), h * D, jnp.int32)

                def _msg(d, carry):
                    colv, colm = carry
                    vw = plsc.load_gather(kv_v, [rows, colv])
                    v0, v1 = _unp(vw)
                    plsc.store_scatter(m_v, [rows, colm], v0 * ex)
                    plsc.store_scatter(m_v, [rows, colm + 1], v1 * ex)
                    return (colv + 1, colm + 2)

                lax.fori_loop(0, W2, _msg, (colv0, colm0), unroll=8)

        pltpu.async_copy(m_v, acc_sh.at[dst_v], ssem, add=True)
        _exslice(ck, xsem).start()
        @pl.when(ck + 1 < NCH)
        def _():
            _eslice(src_hbm, ck + 1, src_v, esem).start()
            _eslice(et_hbm, ck + 1, et_v, esem).start()
        return 0

    lax.fori_loop(0, NCH, _chunk, 0)
    pltpu.make_async_copy(m_v, acc_sh.at[dst_v], ssem).wait()
    _exslice(NCH - 1, xsem).wait()
    plsc.subcore_barrier()
    pltpu.sync_copy(acc_sh.at[rows_sl], msg_hbm.at[cid].at[rows_sl])


KVW2 = HD // 2


def _edge_phase(src, dst, etype, kv_rows, q_rows):
    mesh = plsc.VectorSubcoreMesh(core_axis_name="c", subcore_axis_name="s")
    f = functools.partial(
        pl.kernel,
        out_type=(jax.ShapeDtypeStruct((2, NPAD, HD), _f32),
                  jax.ShapeDtypeStruct((E * H,), _f32)),
        mesh=mesh,
        compiler_params=pltpu.CompilerParams(needs_layout_passes=False),
        scratch_types=[
            pltpu.VMEM_SHARED((NPAD, HD), _f32),
            pltpu.VMEM((CH,), jnp.int32),
            pltpu.VMEM((CH,), jnp.int32),
            pltpu.VMEM((CH,), jnp.int32),
            pltpu.VMEM((CH,), jnp.int32),
            pltpu.VMEM((CH, HD), jnp.int32),
            pltpu.VMEM((CH, HD), _f32),
            pltpu.VMEM((CH, HD), _f32),
            pltpu.VMEM((CH * H,), _f32),
            pltpu.SemaphoreType.DMA,
            pltpu.SemaphoreType.DMA,
            pltpu.SemaphoreType.DMA,
            pltpu.SemaphoreType.DMA,
            pltpu.SemaphoreType.DMA,
        ],
    )(_edge_body)
    return f(src, dst, etype, kv_rows, q_rows)


def _den_body(dst_hbm, ex_hbm, den_hbm, den_v, dst_v, ex_v, dsem, xsem):
    cid = lax.axis_index("c")
    sid = lax.axis_index("s")
    wid = cid * 16 + sid
    iota16 = lax.iota(jnp.int32, 16)
    rep4 = lax.shift_right_logical(iota16, 2)
    c4 = lax.bitwise_and(iota16, 3)

    def _dzero(i, _):
        den_v[pl.ds(i * 16, 16)] = jnp.zeros((16,), _f32)
        return 0

    lax.fori_loop(0, DENW // 16, _dzero, 0)

    def _eslice(ck):
        e0 = wid * EPW + ck * CH
        return pltpu.make_async_copy(dst_hbm.at[pl.ds(e0, CH)], dst_v, dsem)

    def _exslice(ck):
        e0 = wid * EPW + ck * CH
        return pltpu.make_async_copy(ex_hbm.at[pl.ds(e0 * H, CH * H)],
                                     ex_v, xsem)

    _eslice(0).start()
    _exslice(0).start()

    def _chunk(ck, _):
        _eslice(ck).wait()
        _exslice(ck).wait()
        for g in range(CH // 16):
            dw = dst_v[pl.ds(g * 16, 16)]
            for sub in range(4):
                d4 = lax.gather(dw, (sub * 4 + rep4)[:, None],
                                lax.GatherDimensionNumbers(
                                    offset_dims=(),
                                    collapsed_slice_dims=(0,),
                                    start_index_map=(0,)),
                                (1,), mode=lax.GatherScatterMode.PROMISE_IN_BOUNDS)
                exw = ex_v[pl.ds(g * 64 + sub * 16, 16)]
                plsc.addupdate_scatter(den_v, [d4 * H + c4], exw)
        @pl.when(ck + 1 < NCH)
        def _():
            _eslice(ck + 1).start()
            _exslice(ck + 1).start()
        return 0

    lax.fori_loop(0, NCH, _chunk, 0)
    pltpu.sync_copy(den_v, den_hbm.at[wid])


def _den_phase(dst, ex):
    mesh = plsc.VectorSubcoreMesh(core_axis_name="c", subcore_axis_name="s")
    f = functools.partial(
        pl.kernel,
        out_type=jax.ShapeDtypeStruct((NWORK, DENW), _f32),
        mesh=mesh,
        compiler_params=pltpu.CompilerParams(needs_layout_passes=False),
        scratch_types=[
            pltpu.VMEM((DENW,), _f32),
            pltpu.VMEM((CH,), jnp.int32),
            pltpu.VMEM((CH * H,), _f32),
            pltpu.SemaphoreType.DMA,
            pltpu.SemaphoreType.DMA,
        ],
    )(_den_body)
    return f(dst, ex)


def _final_body(a0_ref, a1_ref, dpart_ref, pre_ref, x_ref, nt_ref, wa_ref,
                skip_ref, out_ref):
    acc = a0_ref[...] + a1_ref[...]
    den = jnp.sum(dpart_ref[...], axis=0)
    dfull = jnp.broadcast_to(den[:, :, None], (_RBLK, H, D)).reshape(_RBLK, HD)
    live = dfull > 0.0
    agg = acc / jnp.where(live, dfull, 1.0)
    h1 = LAM * agg + (1.0 - LAM) * pre_ref[...]
    h1 = jnp.where(live, h1, 0.0)
    nt = nt_ref[...]
    x = x_ref[...]
    o = jnp.zeros((_RBLK, HD), dtype=_f32)
    for t in range(NT):
        p = jnp.dot(h1, wa_ref[t], preferred_element_type=_f32)
        alpha = jax.nn.sigmoid(skip_ref[0, t])
        o = o + jnp.where(nt == t, p * alpha + x * (1.0 - alpha), 0.0)
    out_ref[...] = o


def _finalize(acc0, acc1, dparts, pre, x, ntype2d, Wa, skip2d):
    grid = N // _RBLK
    return pl.pallas_call(
        _final_body,
        grid=(grid,),
        in_specs=[
            pl.BlockSpec((_RBLK, HD), lambda i: (i, 0)),
            pl.BlockSpec((_RBLK, HD), lambda i: (i, 0)),
            pl.BlockSpec((NWORK, _RBLK, H), lambda i: (0, i, 0)),
            pl.BlockSpec((_RBLK, HD), lambda i: (i, 0)),
            pl.BlockSpec((_RBLK, IN), lambda i: (i, 0)),
            pl.BlockSpec((_RBLK, 1), lambda i: (i, 0)),
            pl.BlockSpec((NT, HD, HD), lambda i: (0, 0, 0)),
            pl.BlockSpec((1, NT), lambda i: (0, 0)),
        ],
        out_specs=pl.BlockSpec((_RBLK, HD), lambda i: (i, 0)),
        out_shape=jax.ShapeDtypeStruct((N, HD), _f32),
    )(acc0, acc1, dparts, pre, x, ntype2d, Wa, skip2d)


def kernel(x, edge_index, ntype, etype, Wk, Wq, Wv, Wa, Wpre, rel_att,
           rel_msg, rel_pri, skip):
    eye = jnp.eye(H, dtype=_f32)
    att_scaled = rel_att * (rel_pri / SQRT_D)[:, :, None, None]

    def _blockdiag(rel):
        b = eye[None, :, None, :, None] * rel.transpose(1, 2, 0, 3)[:, None, :, :, :]
        return (b.reshape(ET, HD, HD).transpose(1, 0, 2).reshape(HD, ET * HD))

    big_att = _blockdiag(att_scaled)
    big_msg = _blockdiag(rel_msg)

    ntype2d = ntype.reshape(N, 1)
    wbig = _prep_weights(Wk, Wv, Wq, Wpre, big_att, big_msg)
    kv, qb, pre = _project(x, ntype2d, wbig)

    kv_words = lax.bitcast_convert_type(
        kv.reshape(N * ET, 2 * HD // 2, 2), jnp.int32)
    msg, ex = _edge_phase(edge_index[0], edge_index[1], etype,
                          kv_words, qb)
    den = _den_phase(edge_index[1], ex)

    dparts = den.reshape(NWORK, NPAD, H)[:, :N]
    return _finalize(msg[0, :N], msg[1, :N], dparts, pre, x, ntype2d, Wa,
                     skip.reshape(1, NT))


# speedup vs baseline: 3.0664x; 3.0664x over previous
import functools

import jax
import jax.numpy as jnp
import numpy as np
from jax import lax
from jax.experimental import pallas as pl
from jax.experimental.pallas import tpu as pltpu
from jax.experimental.pallas import tpu_sc as plsc

N = 10000
E = 320000
IN = 128
H = 4
D = 32
HD = H * D
NT = 4
ET = 8
SQRT_D = float(np.sqrt(D))
LAM = 0.3

KW = ET * HD
WBW = 2 * KW + 2 * HD
ACCW = 144

NWORK = 32
EPW = E // NWORK
CH = 80
NCH = EPW // CH
NPAD = 10240
ROWS_PT = NPAD // 16
ZR = 128

_f32 = jnp.float32


def _prep_body(wk_ref, wv_ref, wq_ref, wpre_ref, ba_ref, bm_ref, out_ref):
    for t in range(NT):
        out_ref[t, :, 0:KW] = jnp.dot(
            wk_ref[t], ba_ref[...], preferred_element_type=_f32)
        out_ref[t, :, KW:2 * KW] = jnp.dot(
            wv_ref[t], bm_ref[...], preferred_element_type=_f32)
        out_ref[t, :, 2 * KW:2 * KW + HD] = wq_ref[t]
        out_ref[t, :, 2 * KW + HD:WBW] = wpre_ref[t]


def _prep_weights(Wk, Wv, Wq, Wpre, big_att, big_msg):
    return pl.pallas_call(
        _prep_body,
        out_shape=jax.ShapeDtypeStruct((NT, IN, WBW), _f32),
    )(Wk, Wv, Wq, Wpre, big_att, big_msg)


_RBLK = 400


def _proj_body(x_ref, nt_ref, w_ref, khat_ref, vhat_ref, q_ref, pre_ref):
    x = x_ref[...]
    nt = nt_ref[...]
    acc = jnp.zeros((_RBLK, WBW), dtype=_f32)
    for t in range(NT):
        p = jnp.dot(x, w_ref[t], preferred_element_type=_f32)
        acc = acc + jnp.where(nt == t, p, 0.0)
    khat_ref[...] = acc[:, 0:KW]
    vhat_ref[...] = acc[:, KW:2 * KW]
    q_ref[...] = acc[:, 2 * KW:2 * KW + HD]
    pre_ref[...] = acc[:, 2 * KW + HD:WBW]


def _project(x, ntype2d, wbig):
    grid = N // _RBLK
    return pl.pallas_call(
        _proj_body,
        grid=(grid,),
        in_specs=[
            pl.BlockSpec((_RBLK, IN), lambda i: (i, 0)),
            pl.BlockSpec((_RBLK, 1), lambda i: (i, 0)),
            pl.BlockSpec((NT, IN, WBW), lambda i: (0, 0, 0)),
        ],
        out_specs=[
            pl.BlockSpec((_RBLK, KW), lambda i: (i, 0)),
            pl.BlockSpec((_RBLK, KW), lambda i: (i, 0)),
            pl.BlockSpec((_RBLK, HD), lambda i: (i, 0)),
            pl.BlockSpec((_RBLK, HD), lambda i: (i, 0)),
        ],
        out_shape=[
            jax.ShapeDtypeStruct((N, KW), _f32),
            jax.ShapeDtypeStruct((N, KW), _f32),
            jax.ShapeDtypeStruct((N, HD), _f32),
            jax.ShapeDtypeStruct((N, HD), _f32),
        ],
    )(x, ntype2d, wbig)


DENW = NPAD * H


def _edge_body(src_hbm, dst_hbm, et_hbm, khat_hbm, q_hbm, vhat_hbm,
               msg_hbm, ex_hbm,
               acc_sh, src_v, dst_v, et_v, idx_v, khat_v, q_v, vhat_v, m_v,
               ex_v, esem, dsem, gsem, vsem, ssem, xsem):
    cid = lax.axis_index("c")
    sid = lax.axis_index("s")
    wid = cid * 16 + sid
    iota16 = lax.iota(jnp.int32, 16)
    rows_sl = pl.ds(sid * ROWS_PT, ROWS_PT)

    def _zero_mv(r, _):
        for c in range(HD // 16):
            m_v[r, pl.ds(c * 16, 16)] = jnp.zeros((16,), _f32)
        return 0

    lax.fori_loop(0, CH, _zero_mv, 0)
    for j in range(ROWS_PT // CH):
        pltpu.sync_copy(m_v, acc_sh.at[pl.ds(sid * ROWS_PT + j * CH, CH)])
    plsc.subcore_barrier()

    def _eslice(hbm, ck, buf, sem):
        e0 = wid * EPW + ck * CH
        return pltpu.make_async_copy(hbm.at[pl.ds(e0, CH)], buf, sem)

    def _exslice(ck, sem):
        e0 = wid * EPW + ck * CH
        return pltpu.make_async_copy(ex_v, ex_hbm.at[pl.ds(e0 * H, CH * H)],
                                     sem)

    _eslice(src_hbm, 0, src_v, esem).start()
    _eslice(et_hbm, 0, et_v, esem).start()

    def _chunk(ck, _):
        @pl.when(ck > 0)
        def _():
            pltpu.make_async_copy(m_v, acc_sh.at[dst_v], ssem).wait()
            _exslice(ck - 1, xsem).wait()
        _eslice(dst_hbm, ck, dst_v, dsem).start()
        _eslice(src_hbm, ck, src_v, esem).wait()
        _eslice(et_hbm, ck, et_v, esem).wait()
        for g in range(CH // 16):
            s = src_v[pl.ds(g * 16, 16)]
            t = et_v[pl.ds(g * 16, 16)]
            idx_v[pl.ds(g * 16, 16)] = s * ET + t
        pltpu.async_copy(khat_hbm.at[idx_v], khat_v, gsem)
        pltpu.async_copy(vhat_hbm.at[idx_v], vhat_v, vsem)
        _eslice(dst_hbm, ck, dst_v, dsem).wait()
        pltpu.async_copy(q_hbm.at[dst_v], q_v, gsem)
        pltpu.make_async_copy(khat_hbm.at[idx_v], khat_v, gsem).wait()
        pltpu.make_async_copy(q_hbm.at[dst_v], q_v, gsem).wait()
        pltpu.make_async_copy(vhat_hbm.at[idx_v], vhat_v, vsem).wait()

        iotaH = iota16 * H
        z16 = jnp.zeros((16,), _f32)
        for g in range(CH // 16):
            rows = g * 16 + iota16
            for h in range(H):
                col0 = jnp.full((16,), h * D, jnp.int32)

                def _dot(d, carry):
                    a0, a1, col = carry
                    kv0 = plsc.load_gather(khat_v, [rows, col])
                    qv0 = plsc.load_gather(q_v, [rows, col])
                    colh = col + 16
                    kv1 = plsc.load_gather(khat_v, [rows, colh])
                    qv1 = plsc.load_gather(q_v, [rows, colh])
                    return (a0 + kv0 * qv0, a1 + kv1 * qv1, col + 1)

                a0, a1, _ = lax.fori_loop(0, D // 2, _dot, (z16, z16, col0),
                                          unroll=8)
                ex = jnp.exp(jnp.minimum(a0 + a1, 60.0))
                plsc.store_scatter(ex_v, [iotaH + (g * 16 * H + h)], ex)

                def _msg(d, col):
                    vv0 = plsc.load_gather(vhat_v, [rows, col])
                    colh = col + 16
                    vv1 = plsc.load_gather(vhat_v, [rows, colh])
                    plsc.store_scatter(m_v, [rows, col], vv0 * ex)
                    plsc.store_scatter(m_v, [rows, colh], vv1 * ex)
                    return col + 1

                lax.fori_loop(0, D // 2, _msg, col0, unroll=8)

        pltpu.async_copy(m_v, acc_sh.at[dst_v], ssem, add=True)
        _exslice(ck, xsem).start()
        @pl.when(ck + 1 < NCH)
        def _():
            _eslice(src_hbm, ck + 1, src_v, esem).start()
            _eslice(et_hbm, ck + 1, et_v, esem).start()
        return 0

    lax.fori_loop(0, NCH, _chunk, 0)
    pltpu.make_async_copy(m_v, acc_sh.at[dst_v], ssem).wait()
    _exslice(NCH - 1, xsem).wait()
    plsc.subcore_barrier()
    pltpu.sync_copy(acc_sh.at[rows_sl], msg_hbm.at[cid].at[rows_sl])


def _edge_phase(src, dst, etype, khat_rows, q, vhat_rows):
    mesh = plsc.VectorSubcoreMesh(core_axis_name="c", subcore_axis_name="s")
    f = functools.partial(
        pl.kernel,
        out_type=(jax.ShapeDtypeStruct((2, NPAD, HD), _f32),
                  jax.ShapeDtypeStruct((E * H,), _f32)),
        mesh=mesh,
        compiler_params=pltpu.CompilerParams(needs_layout_passes=False),
        scratch_types=[
            pltpu.VMEM_SHARED((NPAD, HD), _f32),
            pltpu.VMEM((CH,), jnp.int32),
            pltpu.VMEM((CH,), jnp.int32),
            pltpu.VMEM((CH,), jnp.int32),
            pltpu.VMEM((CH,), jnp.int32),
            pltpu.VMEM((CH, HD), _f32),
            pltpu.VMEM((CH, HD), _f32),
            pltpu.VMEM((CH, HD), _f32),
            pltpu.VMEM((CH, HD), _f32),
            pltpu.VMEM((CH * H,), _f32),
            pltpu.SemaphoreType.DMA,
            pltpu.SemaphoreType.DMA,
            pltpu.SemaphoreType.DMA,
            pltpu.SemaphoreType.DMA,
            pltpu.SemaphoreType.DMA,
            pltpu.SemaphoreType.DMA,
        ],
    )(_edge_body)
    return f(src, dst, etype, khat_rows, q, vhat_rows)


def _den_body(dst_hbm, ex_hbm, den_hbm, den_v, dst_v, ex_v, dsem, xsem):
    cid = lax.axis_index("c")
    sid = lax.axis_index("s")
    wid = cid * 16 + sid
    iota16 = lax.iota(jnp.int32, 16)
    rep4 = lax.shift_right_logical(iota16, 2)
    c4 = lax.bitwise_and(iota16, 3)

    def _dzero(i, _):
        den_v[pl.ds(i * 16, 16)] = jnp.zeros((16,), _f32)
        return 0

    lax.fori_loop(0, DENW // 16, _dzero, 0)

    def _eslice(ck):
        e0 = wid * EPW + ck * CH
        return pltpu.make_async_copy(dst_hbm.at[pl.ds(e0, CH)], dst_v, dsem)

    def _exslice(ck):
        e0 = wid * EPW + ck * CH
        return pltpu.make_async_copy(ex_hbm.at[pl.ds(e0 * H, CH * H)],
                                     ex_v, xsem)

    _eslice(0).start()
    _exslice(0).start()

    def _chunk(ck, _):
        _eslice(ck).wait()
        _exslice(ck).wait()
        for g in range(CH // 16):
            dw = dst_v[pl.ds(g * 16, 16)]
            for sub in range(4):
                d4 = lax.gather(dw, (sub * 4 + rep4)[:, None],
                                lax.GatherDimensionNumbers(
                                    offset_dims=(),
                                    collapsed_slice_dims=(0,),
                                    start_index_map=(0,)),
                                (1,), mode=lax.GatherScatterMode.PROMISE_IN_BOUNDS)
                exw = ex_v[pl.ds(g * 64 + sub * 16, 16)]
                plsc.addupdate_scatter(den_v, [d4 * H + c4], exw)
        @pl.when(ck + 1 < NCH)
        def _():
            _eslice(ck + 1).start()
            _exslice(ck + 1).start()
        return 0

    lax.fori_loop(0, NCH, _chunk, 0)
    pltpu.sync_copy(den_v, den_hbm.at[wid])


def _den_phase(dst, ex):
    mesh = plsc.VectorSubcoreMesh(core_axis_name="c", subcore_axis_name="s")
    f = functools.partial(
        pl.kernel,
        out_type=jax.ShapeDtypeStruct((NWORK, DENW), _f32),
        mesh=mesh,
        compiler_params=pltpu.CompilerParams(needs_layout_passes=False),
        scratch_types=[
            pltpu.VMEM((DENW,), _f32),
            pltpu.VMEM((CH,), jnp.int32),
            pltpu.VMEM((CH * H,), _f32),
            pltpu.SemaphoreType.DMA,
            pltpu.SemaphoreType.DMA,
        ],
    )(_den_body)
    return f(dst, ex)


def _final_body(a0_ref, a1_ref, dpart_ref, pre_ref, x_ref, nt_ref, wa_ref,
                skip_ref, out_ref):
    acc = a0_ref[...] + a1_ref[...]
    den = jnp.sum(dpart_ref[...], axis=0)
    dfull = jnp.broadcast_to(den[:, :, None], (_RBLK, H, D)).reshape(_RBLK, HD)
    live = dfull > 0.0
    agg = acc / jnp.where(live, dfull, 1.0)
    h1 = LAM * agg + (1.0 - LAM) * pre_ref[...]
    h1 = jnp.where(live, h1, 0.0)
    nt = nt_ref[...]
    x = x_ref[...]
    o = jnp.zeros((_RBLK, HD), dtype=_f32)
    for t in range(NT):
        p = jnp.dot(h1, wa_ref[t], preferred_element_type=_f32)
        alpha = jax.nn.sigmoid(skip_ref[0, t])
        o = o + jnp.where(nt == t, p * alpha + x * (1.0 - alpha), 0.0)
    out_ref[...] = o


def _finalize(acc0, acc1, dparts, pre, x, ntype2d, Wa, skip2d):
    grid = N // _RBLK
    return pl.pallas_call(
        _final_body,
        grid=(grid,),
        in_specs=[
            pl.BlockSpec((_RBLK, HD), lambda i: (i, 0)),
            pl.BlockSpec((_RBLK, HD), lambda i: (i, 0)),
            pl.BlockSpec((NWORK, _RBLK, H), lambda i: (0, i, 0)),
            pl.BlockSpec((_RBLK, HD), lambda i: (i, 0)),
            pl.BlockSpec((_RBLK, IN), lambda i: (i, 0)),
            pl.BlockSpec((_RBLK, 1), lambda i: (i, 0)),
            pl.BlockSpec((NT, HD, HD), lambda i: (0, 0, 0)),
            pl.BlockSpec((1, NT), lambda i: (0, 0)),
        ],
        out_specs=pl.BlockSpec((_RBLK, HD), lambda i: (i, 0)),
        out_shape=jax.ShapeDtypeStruct((N, HD), _f32),
    )(acc0, acc1, dparts, pre, x, ntype2d, Wa, skip2d)


def kernel(x, edge_index, ntype, etype, Wk, Wq, Wv, Wa, Wpre, rel_att,
           rel_msg, rel_pri, skip):
    eye = jnp.eye(H, dtype=_f32)
    att_scaled = rel_att * (rel_pri / SQRT_D)[:, :, None, None]

    def _blockdiag(rel):
        b = eye[None, :, None, :, None] * rel.transpose(1, 2, 0, 3)[:, None, :, :, :]
        return (b.reshape(ET, HD, HD).transpose(1, 0, 2).reshape(HD, ET * HD))

    big_att = _blockdiag(att_scaled)
    big_msg = _blockdiag(rel_msg)

    ntype2d = ntype.reshape(N, 1)
    wbig = _prep_weights(Wk, Wv, Wq, Wpre, big_att, big_msg)
    khat, vhat, q, pre = _project(x, ntype2d, wbig)

    msg, ex = _edge_phase(edge_index[0], edge_index[1], etype,
                          khat.reshape(N * ET, HD), q,
                          vhat.reshape(N * ET, HD))
    den = _den_phase(edge_index[1], ex)

    dparts = den.reshape(NWORK, NPAD, H)[:, :N]
    return _finalize(msg[0, :N], msg[1, :N], dparts, pre, x, ntype2d, Wa,
                     skip.reshape(1, NT))
